# baseline (speedup 1.0000x reference)
import functools

import jax
import jax.numpy as jnp
from jax import lax
from jax.experimental import pallas as pl
from jax.experimental.pallas import tpu as pltpu
from jax.experimental.pallas import tpu_sc as plsc


def _build(batch: int, seq: int, hid: int, nb: int):
    info = plsc.get_sparse_core_info()
    nc, ns = info.num_cores, info.num_subcores
    nw = nc * ns
    assert batch % (nw * nb) == 0
    seqs_per_w = batch // nw
    n_chunks = seqs_per_w // nb
    assert hid % 16 == 0
    nh = hid // 16

    mesh = plsc.VectorSubcoreMesh(core_axis_name="c", subcore_axis_name="s")

    @functools.partial(
        pl.kernel,
        mesh=mesh,
        compiler_params=pltpu.CompilerParams(use_tc_tiling_on_sc=False),
        out_type=jax.ShapeDtypeStruct((batch, seq, hid), jnp.float32),
        scratch_types=[
            pltpu.VMEM((nb, seq), jnp.int32),
            pltpu.VMEM((nb, seq, hid), jnp.float32),
            pltpu.VMEM((seq, hid), jnp.float32),
            pltpu.SemaphoreType.DMA,
        ],
    )
    def emb_lookup(x_hbm, emb_hbm, pos_hbm, out_hbm, idx_v, rows_v, pos_v, sem):
        wid = lax.axis_index("s") * nc + lax.axis_index("c")
        base = wid * seqs_per_w
        pltpu.sync_copy(pos_hbm, pos_v)
        for g in range(n_chunks):
            b0 = base + g * nb
            pltpu.sync_copy(x_hbm.at[pl.ds(b0, nb)], idx_v)
            copies = [
                pltpu.async_copy(emb_hbm.at[idx_v.at[j]], rows_v.at[j], sem)
                for j in range(nb)
            ]
            for c in copies:
                c.wait()

            def add_pos(s, carry):
                for j in range(nb):
                    for h in range(nh):
                        sl = pl.ds(h * 16, 16)
                        rows_v[j, s, sl] = rows_v[j, s, sl] + pos_v[s, sl]
                return carry

            lax.fori_loop(0, seq, add_pos, 0)
            pltpu.sync_copy(rows_v, out_hbm.at[pl.ds(b0, nb)])

    return emb_lookup


def kernel(x, emb, pos_emb):
    b, s = x.shape
    hid = emb.shape[1]
    fn = _build(b, s, hid, nb=8)
    return fn(x.astype(jnp.int32), emb, pos_emb)
